# baseline (speedup 1.0000x reference)
import numpy as np

import jax
import jax.numpy as jnp
from jax import lax
from jax.experimental import pallas as pl
from jax.experimental.pallas import tpu as pltpu
from jax.experimental.pallas import tpu_sc as plsc

B = 128
V = 100000
NVREG = V // 16
NC, NS, L = 2, 16, 16
NW = NC * NS
ROWS_PER_W = B // NW
NEED = 99
CAP = 512
NSORT = 112
NEGW = 20000
BLK = 10
NBLK = NVREG // BLK
NEG = float(jnp.finfo(jnp.float32).min)
HIBIT = np.uint32(0x80000000)


def _key_of(v):
    u = lax.bitcast_convert_type(v, jnp.uint32)
    sa = lax.shift_right_arithmetic(lax.bitcast_convert_type(v, jnp.int32), 31)
    return u ^ (lax.bitcast_convert_type(sa, jnp.uint32) | HIBIT)


def _key_scalar(v):
    u = lax.bitcast_convert_type(v, jnp.uint32)
    sa = lax.shift_right_arithmetic(lax.bitcast_convert_type(v, jnp.int32), 31)
    return u ^ (lax.bitcast_convert_type(sa, jnp.uint32) | HIBIT)


def _scan_hist(hist_ref, nbuckets, cg):
    nv = nbuckets // 16
    lanes = lax.iota(jnp.int32, 16)

    def body(i, carry):
        acc, chosen, cnt, found = carry
        t = nv - 1 - i
        h = hist_ref[pl.ds(t * 16, 16)]
        tot = jnp.sum(h)
        crossing = jnp.logical_and(jnp.logical_not(found),
                                   cg + acc + tot >= NEED)
        cum = plsc.cumsum(h)
        suff = tot - cum + h
        cond = (cg + acc + suff) >= NEED
        lane = jnp.max(jnp.where(cond, lanes, -1))
        lane_c = jnp.maximum(lane, 0)
        onlane = lanes == lane_c
        h_at = jnp.max(jnp.where(onlane, h, 0))
        cum_at = jnp.max(jnp.where(onlane, cum, 0))
        acc_new = jnp.where(found, acc,
                            jnp.where(crossing, acc + (tot - cum_at),
                                      acc + tot))
        chosen = jnp.where(crossing, t * 16 + lane_c, chosen)
        cnt = jnp.where(crossing, h_at, cnt)
        found = jnp.logical_or(found, crossing)
        return acc_new, chosen, cnt, found

    acc, chosen, cnt, _ = lax.fori_loop(
        0, nv, body, (jnp.int32(0), jnp.int32(0), jnp.int32(0),
                      jnp.bool_(False)))
    cg_new = cg + acc
    return chosen, cg_new, cg_new + cnt


def _scan_hist_h(hist_ref, tot_ref, nbuckets, cg):
    nv = nbuckets // 16
    lanes = lax.iota(jnp.int32, 16)

    @plsc.parallel_loop(0, nv, unroll=8)
    def _(i):
        h = hist_ref[pl.ds(i * 16, 16)]
        plsc.store_scatter(tot_ref, [jnp.full((16,), 0, jnp.int32) + i],
                           jnp.zeros((16,), jnp.int32) + jnp.sum(h),
                           mask=lanes == 0)

    tv, cgv, _ = _scan_hist(tot_ref, nv, cg)
    h = hist_ref[pl.ds(tv * 16, 16)]
    tot = jnp.sum(h)
    cum = plsc.cumsum(h)
    suff = tot - cum + h
    cond = (cgv + suff) >= NEED
    lane = jnp.maximum(jnp.max(jnp.where(cond, lanes, -1)), 0)
    onlane = lanes == lane
    h_at = jnp.max(jnp.where(onlane, h, 0))
    cum_at = jnp.max(jnp.where(onlane, cum, 0))
    cg_new = cgv + (tot - cum_at)
    return tv * 16 + lane, cg_new, cg_new + h_at


def _zero_hist(hist_ref):
    zeros = jnp.zeros((16,), jnp.int32)

    @plsc.parallel_loop(0, 2048 // 16, unroll=8)
    def _(i):
        hist_ref[pl.ds(i * 16, 16)] = zeros


def _body(logits_hbm, q_hbm, topk_hbm, tpeps_hbm,
          tok_hbm, out_hbm,
          row_ref, hist_ref, ck_ref, ci_ref,
          sv_ref, si_ref, eb_ref, pb_ref, rb_ref,
          tk_ref, tp_ref, tokbuf_ref, bm_ref, tot_ref, sem):
    wid = lax.axis_index("s") * NC + lax.axis_index("c")
    lanes = lax.iota(jnp.int32, 16)
    ones = jnp.ones((16,), jnp.int32)

    pltpu.sync_copy(topk_hbm, tk_ref)
    pltpu.sync_copy(tpeps_hbm, tp_ref)
    tokbuf_ref[pl.ds(0, 16)] = jnp.zeros((16,), jnp.int32)

    negv = jnp.full((16,), NEG, jnp.float32)

    def do_row(j, _):
        row = wid * ROWS_PER_W + j
        pltpu.sync_copy(logits_hbm.at[row], row_ref)
        _zero_hist(hist_ref)

        def p1b(bi, _):
            base = bi * BLK

            @plsc.parallel_loop(0, BLK, unroll=10,
                                carry=jnp.zeros((16,), jnp.uint32))
            def mx(i, bm):
                v = row_ref[pl.ds((base + i) * 16, 16)]
                k = _key_of(v)
                row_ref[pl.ds((base + i) * 16, 16)] = (
                    lax.bitcast_convert_type(k, jnp.float32))
                b = lax.bitcast_convert_type(k >> 21, jnp.int32)
                plsc.addupdate_scatter(hist_ref, [b], ones)
                return jnp.maximum(bm, k)

            bms = jnp.max(lax.bitcast_convert_type(mx ^ HIBIT, jnp.int32))
            plsc.store_scatter(bm_ref,
                               [jnp.full((16,), 0, jnp.int32) + bi],
                               jnp.zeros((16,), jnp.int32) + bms,
                               mask=lanes == 0)
            return 0

        lax.fori_loop(0, NBLK, p1b, 0)
        b0, cg, count_ge = _scan_hist_h(hist_ref, tot_ref, 2048,
                                        jnp.int32(0))
        T = lax.convert_element_type(b0, jnp.uint32) << 21

        def _kld(i):
            return lax.bitcast_convert_type(row_ref[pl.ds(i * 16, 16)],
                                            jnp.uint32)

        def refine(level_shift, level_mask, prev_shift, T, cg):
            _zero_hist(hist_ref)
            Tp = T >> prev_shift
            pmin_s = lax.bitcast_convert_type((Tp << prev_shift) ^ HIBIT,
                                              jnp.int32)

            def rblk(bi, _):
                bms = bm_ref[pl.ds(bi, 16)][0]
                base = bi * BLK

                def go(z):
                    @plsc.parallel_loop(0, BLK, unroll=10)
                    def _(i):
                        k = _kld(base + i)
                        part = (k >> prev_shift) == Tp
                        b = lax.bitcast_convert_type(
                            (k >> level_shift) & jnp.uint32(level_mask),
                            jnp.int32)
                        plsc.addupdate_scatter(hist_ref, [b], ones,
                                               mask=part)
                    return z

                return lax.cond(bms >= pmin_s, go, lambda z: z, 0)

            lax.fori_loop(0, NBLK, rblk, 0)
            bb, cg2, cge = _scan_hist(hist_ref, level_mask + 1, cg)
            T2 = T | (lax.convert_element_type(bb, jnp.uint32) << level_shift)
            return T2, cg2, cge

        def lvl1(args):
            T, cg, _ = args
            return refine(10, 0x7FF, 21, T, cg)

        T, cg, count_ge = lax.cond(count_ge <= CAP,
                                   lambda a: a, lvl1, (T, cg, count_ge))

        def lvl2(args):
            T, cg, _ = args
            return refine(0, 0x3FF, 10, T, cg)

        T, cg, count_ge = lax.cond(count_ge <= CAP,
                                   lambda a: a, lvl2, (T, cg, count_ge))

        Ts = lax.bitcast_convert_type(T ^ HIBIT, jnp.int32)

        def lvlI(args):
            T, cg, _ = args
            _zero_hist(hist_ref)

            def iblk(bi, _):
                bms = bm_ref[pl.ds(bi, 16)][0]
                base = bi * BLK

                def go(z):
                    @plsc.parallel_loop(0, BLK, unroll=10)
                    def _(i):
                        k = _kld(base + i)
                        part = k == T
                        idx = lanes + (base + i) * 16
                        b = lax.shift_right_logical(idx, 8)
                        plsc.addupdate_scatter(hist_ref, [b], ones,
                                               mask=part)
                    return z

                return lax.cond(bms >= Ts, go, lambda z: z, 0)

            lax.fori_loop(0, NBLK, iblk, 0)
            bb, _, _ = _scan_hist(hist_ref, 512, cg)
            return lax.shift_left(bb, 8)

        I = lax.cond(count_ge <= CAP, lambda a: jnp.int32(0), lvlI,
                     (T, cg, count_ge))

        def cblk_fast(bi, ptr):
            bms = bm_ref[pl.ds(bi, 16)][0]
            base = bi * BLK

            def go(p):
                @plsc.parallel_loop(0, BLK, unroll=10, carry=p)
                def cc(i, q):
                    k = _kld(base + i)
                    mk = k >= T
                    plsc.store_compressed(
                        ck_ref.at[pl.ds(q, 16)],
                        lax.bitcast_convert_type(k, jnp.int32), mask=mk)
                    plsc.store_compressed(ci_ref.at[pl.ds(q, 16)],
                                          lanes + (base + i) * 16, mask=mk)
                    return q + plsc.all_reduce_population_count(mk)[0]

                return cc

            return lax.cond(bms >= Ts, go, lambda p: p, ptr)

        def cblk_tie(bi, ptr):
            bms = bm_ref[pl.ds(bi, 16)][0]
            base = bi * BLK

            def go(p):
                @plsc.parallel_loop(0, BLK, unroll=10, carry=p)
                def cc(i, q):
                    k = _kld(base + i)
                    idx = lanes + (base + i) * 16
                    mk = jnp.logical_or(k > T,
                                        jnp.logical_and(k == T, idx >= I))
                    plsc.store_compressed(
                        ck_ref.at[pl.ds(q, 16)],
                        lax.bitcast_convert_type(k, jnp.int32), mask=mk)
                    plsc.store_compressed(ci_ref.at[pl.ds(q, 16)], idx,
                                          mask=mk)
                    return q + plsc.all_reduce_population_count(mk)[0]

                return cc

            return lax.cond(bms >= Ts, go, lambda p: p, ptr)

        n = lax.cond(
            I == 0,
            lambda z: lax.fori_loop(0, NBLK, cblk_fast, z),
            lambda z: lax.fori_loop(0, NBLK, cblk_tie, z),
            jnp.int32(0))

        qh = pltpu.async_copy(q_hbm.at[row], row_ref, sem)

        def sinit(t, _):
            sv_ref[pl.ds(t * 16, 16)] = jnp.full((16,), NEG, jnp.float32)
            si_ref[pl.ds(t * 16, 16)] = jnp.zeros((16,), jnp.int32)
            return 0

        lax.fori_loop(0, NSORT // 16, sinit, 0)

        nv_c = lax.shift_right_logical(n + 15, 4)

        def rank_one(i, _):
            ki = lax.convert_element_type(ck_ref[pl.ds(i, 16)][0], jnp.uint32)
            ii = ci_ref[pl.ds(i, 16)][0]

            def rin(jv, r):
                kv = lax.convert_element_type(ck_ref[pl.ds(jv * 16, 16)],
                                              jnp.uint32)
                iv = ci_ref[pl.ds(jv * 16, 16)]
                ok = (jv * 16 + lanes) < n
                gt = jnp.logical_or(kv > ki,
                                    jnp.logical_and(kv == ki, iv > ii))
                return r + plsc.all_reduce_population_count(
                    jnp.logical_and(gt, ok))

            rank_v = lax.fori_loop(0, nv_c, rin, jnp.zeros((16,), jnp.int32))
            rank = rank_v[0]
            neg = (ki >> 31) == 0
            bits = jnp.where(neg, ~ki, ki ^ HIBIT)
            val = lax.bitcast_convert_type(bits, jnp.float32)
            lane0 = jnp.logical_and(lanes == 0, rank < NEED)
            slot_v = jnp.full((16,), 0, jnp.int32) + rank
            plsc.store_scatter(sv_ref, [slot_v],
                               jnp.zeros((16,), jnp.float32) + val,
                               mask=lane0)
            plsc.store_scatter(si_ref, [slot_v],
                               jnp.zeros((16,), jnp.int32) + ii,
                               mask=lane0)
            return 0

        lax.fori_loop(0, n, rank_one, 0)

        k_row = tk_ref[pl.ds(row, 16)][0]
        p_row = tp_ref[pl.ds(row, 16)][0]
        eps = tp_ref[pl.ds(B, 16)][0]
        v0 = sv_ref[pl.ds(0, 16)][0]

        def ebody(t, z):
            sv = sv_ref[pl.ds(t * 16, 16)]
            r = lanes + t * 16
            e = jnp.where(r < k_row, jnp.exp(sv - v0), jnp.float32(0.0))
            eb_ref[pl.ds(t * 16, 16)] = e
            return z + jnp.sum(e)

        Z = lax.fori_loop(0, NSORT // 16, ebody, jnp.float32(0.0))

        def cbody(t, carry):
            cc, m = carry
            e = eb_ref[pl.ds(t * 16, 16)]
            probs = e / Z
            pb_ref[pl.ds(t * 16, 16)] = probs
            cum = plsc.cumsum(probs) + cc
            r = lanes + t * 16
            keep = jnp.logical_and(r < k_row, (cum - probs) <= p_row)
            m = m + jnp.sum(keep.astype(jnp.int32))
            return jnp.max(cum), m

        _, m = lax.fori_loop(0, NSORT // 16, cbody,
                             (jnp.float32(0.0), jnp.int32(0)))

        qh.wait()

        def rbody(t, best):
            r = lanes + t * 16
            probs = pb_ref[pl.ds(t * 16, 16)]
            qv = plsc.load_gather(row_ref, [si_ref[pl.ds(t * 16, 16)]])
            ratio = jnp.where(r < m, probs / (qv + eps), jnp.float32(0.0))
            rb_ref[pl.ds(t * 16, 16)] = ratio
            return jnp.maximum(best, jnp.max(ratio))

        best = lax.fori_loop(0, NSORT // 16, rbody, jnp.float32(0.0))

        def tbody(t, tok):
            r = lanes + t * 16
            ratio = rb_ref[pl.ds(t * 16, 16)]
            iv = si_ref[pl.ds(t * 16, 16)]
            hit = jnp.logical_and(ratio == best, r < m)
            return jnp.minimum(tok, jnp.min(jnp.where(hit, iv, V)))

        tok = lax.fori_loop(0, NSORT // 16, tbody, jnp.int32(V))
        plsc.store_scatter(tokbuf_ref, [jnp.full((16,), 0, jnp.int32) + j],
                           jnp.zeros((16,), jnp.int32) + tok,
                           mask=lanes == 0)

        @plsc.parallel_loop(0, NVREG, unroll=10)
        def _(i):
            row_ref[pl.ds(i * 16, 16)] = negv

        def wbody(t, _):
            r = lanes + t * 16
            plsc.store_scatter(row_ref, [si_ref[pl.ds(t * 16, 16)]],
                               sv_ref[pl.ds(t * 16, 16)], mask=r < m)
            return 0

        lax.fori_loop(0, NSORT // 16, wbody, 0)
        pltpu.sync_copy(row_ref, out_hbm.at[row])
        return 0

    lax.fori_loop(0, ROWS_PER_W, do_row, 0)
    pltpu.sync_copy(tokbuf_ref, tok_hbm.at[wid])


@jax.jit
def kernel(logits, top_k, top_p, q, eps):
    tpeps = jnp.concatenate(
        [top_p.astype(jnp.float32),
         jnp.full((32,), eps, jnp.float32)])
    topk_pad = jnp.concatenate(
        [top_k.astype(jnp.int32), jnp.zeros((32,), jnp.int32)])
    mesh = plsc.VectorSubcoreMesh(core_axis_name="c", subcore_axis_name="s",
                                  num_cores=NC, num_subcores=NS)
    f = pl.kernel(
        _body,
        out_type=[
            jax.ShapeDtypeStruct((NW, 16), jnp.int32),
            jax.ShapeDtypeStruct((B, V), jnp.float32),
        ],
        mesh=mesh,
        compiler_params=pltpu.CompilerParams(needs_layout_passes=False),
        scratch_types=[
            pltpu.VMEM((V,), jnp.float32),
            pltpu.VMEM((2048,), jnp.int32),
            pltpu.VMEM((CAP + 16,), jnp.int32),
            pltpu.VMEM((CAP + 16,), jnp.int32),
            pltpu.VMEM((NSORT + 16,), jnp.float32),
            pltpu.VMEM((NSORT + 16,), jnp.int32),
            pltpu.VMEM((NSORT,), jnp.float32),
            pltpu.VMEM((NSORT,), jnp.float32),
            pltpu.VMEM((NSORT,), jnp.float32),
            pltpu.VMEM((B + 32,), jnp.int32),
            pltpu.VMEM((B + 32,), jnp.float32),
            pltpu.VMEM((16,), jnp.int32),
            pltpu.VMEM((NBLK + 16,), jnp.int32),
            pltpu.VMEM((144,), jnp.int32),
            pltpu.SemaphoreType.DMA,
        ],
    )
    tok_pad, masked = f(logits, q, topk_pad, tpeps)
    tokens = tok_pad[:, :ROWS_PER_W].reshape(B)
    return tokens, masked


# speedup vs baseline: 1.1476x; 1.1476x over previous
import numpy as np

import jax
import jax.numpy as jnp
from jax import lax
from jax.experimental import pallas as pl
from jax.experimental.pallas import tpu as pltpu
from jax.experimental.pallas import tpu_sc as plsc

B = 128
V = 100000
NVREG = V // 16
NC, NS, L = 2, 16, 16
NW = NC * NS
ROWS_PER_W = B // NW
NEED = 99
CAP = 512
NSORT = 112
NEGW = 20000
BLK = 25
NBLK = NVREG // BLK
NEG = float(jnp.finfo(jnp.float32).min)
HIBIT = np.uint32(0x80000000)


def _key_of(v):
    u = lax.bitcast_convert_type(v, jnp.uint32)
    sa = lax.shift_right_arithmetic(lax.bitcast_convert_type(v, jnp.int32), 31)
    return u ^ (lax.bitcast_convert_type(sa, jnp.uint32) | HIBIT)


def _key_scalar(v):
    u = lax.bitcast_convert_type(v, jnp.uint32)
    sa = lax.shift_right_arithmetic(lax.bitcast_convert_type(v, jnp.int32), 31)
    return u ^ (lax.bitcast_convert_type(sa, jnp.uint32) | HIBIT)


def _scan_hist(hist_ref, nbuckets, cg):
    nv = nbuckets // 16
    lanes = lax.iota(jnp.int32, 16)

    def body(i, carry):
        acc, chosen, cnt, found = carry
        t = nv - 1 - i
        h = hist_ref[pl.ds(t * 16, 16)]
        tot = jnp.sum(h)
        crossing = jnp.logical_and(jnp.logical_not(found),
                                   cg + acc + tot >= NEED)
        cum = plsc.cumsum(h)
        suff = tot - cum + h
        cond = (cg + acc + suff) >= NEED
        lane = jnp.max(jnp.where(cond, lanes, -1))
        lane_c = jnp.maximum(lane, 0)
        onlane = lanes == lane_c
        h_at = jnp.max(jnp.where(onlane, h, 0))
        cum_at = jnp.max(jnp.where(onlane, cum, 0))
        acc_new = jnp.where(found, acc,
                            jnp.where(crossing, acc + (tot - cum_at),
                                      acc + tot))
        chosen = jnp.where(crossing, t * 16 + lane_c, chosen)
        cnt = jnp.where(crossing, h_at, cnt)
        found = jnp.logical_or(found, crossing)
        return acc_new, chosen, cnt, found

    acc, chosen, cnt, _ = lax.fori_loop(
        0, nv, body, (jnp.int32(0), jnp.int32(0), jnp.int32(0),
                      jnp.bool_(False)))
    cg_new = cg + acc
    return chosen, cg_new, cg_new + cnt


def _scan_hist_h(hist_ref, tot_ref, nbuckets, cg):
    nv = nbuckets // 16
    lanes = lax.iota(jnp.int32, 16)

    @plsc.parallel_loop(0, nv, unroll=8)
    def _(i):
        h = hist_ref[pl.ds(i * 16, 16)]
        plsc.store_scatter(tot_ref, [jnp.full((16,), 0, jnp.int32) + i],
                           jnp.zeros((16,), jnp.int32) + jnp.sum(h),
                           mask=lanes == 0)

    tv, cgv, _ = _scan_hist(tot_ref, nv, cg)
    h = hist_ref[pl.ds(tv * 16, 16)]
    tot = jnp.sum(h)
    cum = plsc.cumsum(h)
    suff = tot - cum + h
    cond = (cgv + suff) >= NEED
    lane = jnp.maximum(jnp.max(jnp.where(cond, lanes, -1)), 0)
    onlane = lanes == lane
    h_at = jnp.max(jnp.where(onlane, h, 0))
    cum_at = jnp.max(jnp.where(onlane, cum, 0))
    cg_new = cgv + (tot - cum_at)
    return tv * 16 + lane, cg_new, cg_new + h_at


def _zero_hist(hist_ref):
    zeros = jnp.zeros((16,), jnp.int32)

    @plsc.parallel_loop(0, 2048 // 16, unroll=8)
    def _(i):
        hist_ref[pl.ds(i * 16, 16)] = zeros


def _body(logits_hbm, q_hbm, topk_hbm, tpeps_hbm,
          tok_hbm, out_hbm,
          row_ref, hist_ref, ck_ref, ci_ref,
          sv_ref, si_ref, eb_ref, pb_ref, rb_ref,
          tk_ref, tp_ref, tokbuf_ref, bm_ref, tot_ref, sem):
    wid = lax.axis_index("s") * NC + lax.axis_index("c")
    lanes = lax.iota(jnp.int32, 16)
    ones = jnp.ones((16,), jnp.int32)

    pltpu.sync_copy(topk_hbm, tk_ref)
    pltpu.sync_copy(tpeps_hbm, tp_ref)
    tokbuf_ref[pl.ds(0, 16)] = jnp.zeros((16,), jnp.int32)

    negv = jnp.full((16,), NEG, jnp.float32)

    def do_row(j, _):
        row = wid * ROWS_PER_W + j
        pltpu.sync_copy(logits_hbm.at[row], row_ref)
        _zero_hist(hist_ref)

        def p1b(bi, _):
            base = bi * BLK

            @plsc.parallel_loop(0, BLK, unroll=25,
                                carry=jnp.zeros((16,), jnp.uint32))
            def mx(i, bm):
                v = row_ref[pl.ds((base + i) * 16, 16)]
                k = _key_of(v)
                row_ref[pl.ds((base + i) * 16, 16)] = (
                    lax.bitcast_convert_type(k, jnp.float32))
                b = lax.bitcast_convert_type(k >> 21, jnp.int32)
                plsc.addupdate_scatter(hist_ref, [b], ones)
                return jnp.maximum(bm, k)

            bms = jnp.max(lax.bitcast_convert_type(mx ^ HIBIT, jnp.int32))
            plsc.store_scatter(bm_ref,
                               [jnp.full((16,), 0, jnp.int32) + bi],
                               jnp.zeros((16,), jnp.int32) + bms,
                               mask=lanes == 0)
            return 0

        lax.fori_loop(0, NBLK, p1b, 0)
        b0, cg, count_ge = _scan_hist_h(hist_ref, tot_ref, 2048,
                                        jnp.int32(0))
        T = lax.convert_element_type(b0, jnp.uint32) << 21

        def _kld(i):
            return lax.bitcast_convert_type(row_ref[pl.ds(i * 16, 16)],
                                            jnp.uint32)

        def refine(level_shift, level_mask, prev_shift, T, cg):
            _zero_hist(hist_ref)
            Tp = T >> prev_shift
            pmin_s = lax.bitcast_convert_type((Tp << prev_shift) ^ HIBIT,
                                              jnp.int32)

            def rblk(bi, _):
                bms = bm_ref[pl.ds(bi, 16)][0]
                base = bi * BLK

                def go(z):
                    @plsc.parallel_loop(0, BLK, unroll=25)
                    def _(i):
                        k = _kld(base + i)
                        part = (k >> prev_shift) == Tp
                        b = lax.bitcast_convert_type(
                            (k >> level_shift) & jnp.uint32(level_mask),
                            jnp.int32)
                        plsc.addupdate_scatter(hist_ref, [b], ones,
                                               mask=part)
                    return z

                return lax.cond(bms >= pmin_s, go, lambda z: z, 0)

            lax.fori_loop(0, NBLK, rblk, 0)
            bb, cg2, cge = _scan_hist(hist_ref, level_mask + 1, cg)
            T2 = T | (lax.convert_element_type(bb, jnp.uint32) << level_shift)
            return T2, cg2, cge

        def lvl1(args):
            T, cg, _ = args
            return refine(10, 0x7FF, 21, T, cg)

        T, cg, count_ge = lax.cond(count_ge <= CAP,
                                   lambda a: a, lvl1, (T, cg, count_ge))

        def lvl2(args):
            T, cg, _ = args
            return refine(0, 0x3FF, 10, T, cg)

        T, cg, count_ge = lax.cond(count_ge <= CAP,
                                   lambda a: a, lvl2, (T, cg, count_ge))

        Ts = lax.bitcast_convert_type(T ^ HIBIT, jnp.int32)

        def lvlI(args):
            T, cg, _ = args
            _zero_hist(hist_ref)

            def iblk(bi, _):
                bms = bm_ref[pl.ds(bi, 16)][0]
                base = bi * BLK

                def go(z):
                    @plsc.parallel_loop(0, BLK, unroll=25)
                    def _(i):
                        k = _kld(base + i)
                        part = k == T
                        idx = lanes + (base + i) * 16
                        b = lax.shift_right_logical(idx, 8)
                        plsc.addupdate_scatter(hist_ref, [b], ones,
                                               mask=part)
                    return z

                return lax.cond(bms >= Ts, go, lambda z: z, 0)

            lax.fori_loop(0, NBLK, iblk, 0)
            bb, _, _ = _scan_hist(hist_ref, 512, cg)
            return lax.shift_left(bb, 8)

        I = lax.cond(count_ge <= CAP, lambda a: jnp.int32(0), lvlI,
                     (T, cg, count_ge))

        def cblk_fast(bi, ptr):
            bms = bm_ref[pl.ds(bi, 16)][0]
            base = bi * BLK

            def go(p):
                @plsc.parallel_loop(0, BLK, unroll=25, carry=p)
                def cc(i, q):
                    k = _kld(base + i)
                    mk = k >= T
                    plsc.store_compressed(
                        ck_ref.at[pl.ds(q, 16)],
                        lax.bitcast_convert_type(k, jnp.int32), mask=mk)
                    plsc.store_compressed(ci_ref.at[pl.ds(q, 16)],
                                          lanes + (base + i) * 16, mask=mk)
                    return q + plsc.all_reduce_population_count(mk)[0]

                return cc

            return lax.cond(bms >= Ts, go, lambda p: p, ptr)

        def cblk_tie(bi, ptr):
            bms = bm_ref[pl.ds(bi, 16)][0]
            base = bi * BLK

            def go(p):
                @plsc.parallel_loop(0, BLK, unroll=25, carry=p)
                def cc(i, q):
                    k = _kld(base + i)
                    idx = lanes + (base + i) * 16
                    mk = jnp.logical_or(k > T,
                                        jnp.logical_and(k == T, idx >= I))
                    plsc.store_compressed(
                        ck_ref.at[pl.ds(q, 16)],
                        lax.bitcast_convert_type(k, jnp.int32), mask=mk)
                    plsc.store_compressed(ci_ref.at[pl.ds(q, 16)], idx,
                                          mask=mk)
                    return q + plsc.all_reduce_population_count(mk)[0]

                return cc

            return lax.cond(bms >= Ts, go, lambda p: p, ptr)

        n = lax.cond(
            I == 0,
            lambda z: lax.fori_loop(0, NBLK, cblk_fast, z),
            lambda z: lax.fori_loop(0, NBLK, cblk_tie, z),
            jnp.int32(0))

        qh = pltpu.async_copy(q_hbm.at[row], row_ref, sem)

        def sinit(t, _):
            sv_ref[pl.ds(t * 16, 16)] = jnp.full((16,), NEG, jnp.float32)
            si_ref[pl.ds(t * 16, 16)] = jnp.zeros((16,), jnp.int32)
            return 0

        lax.fori_loop(0, NSORT // 16, sinit, 0)

        nv_c = lax.shift_right_logical(n + 15, 4)

        def rank_one(i, _):
            ki = lax.convert_element_type(ck_ref[pl.ds(i, 16)][0], jnp.uint32)
            ii = ci_ref[pl.ds(i, 16)][0]

            def rin(jv, r):
                kv = lax.convert_element_type(ck_ref[pl.ds(jv * 16, 16)],
                                              jnp.uint32)
                iv = ci_ref[pl.ds(jv * 16, 16)]
                ok = (jv * 16 + lanes) < n
                gt = jnp.logical_or(kv > ki,
                                    jnp.logical_and(kv == ki, iv > ii))
                return r + plsc.all_reduce_population_count(
                    jnp.logical_and(gt, ok))

            rank_v = lax.fori_loop(0, nv_c, rin, jnp.zeros((16,), jnp.int32))
            rank = rank_v[0]
            neg = (ki >> 31) == 0
            bits = jnp.where(neg, ~ki, ki ^ HIBIT)
            val = lax.bitcast_convert_type(bits, jnp.float32)
            lane0 = jnp.logical_and(lanes == 0, rank < NEED)
            slot_v = jnp.full((16,), 0, jnp.int32) + rank
            plsc.store_scatter(sv_ref, [slot_v],
                               jnp.zeros((16,), jnp.float32) + val,
                               mask=lane0)
            plsc.store_scatter(si_ref, [slot_v],
                               jnp.zeros((16,), jnp.int32) + ii,
                               mask=lane0)
            return 0

        lax.fori_loop(0, n, rank_one, 0)

        k_row = tk_ref[pl.ds(row, 16)][0]
        p_row = tp_ref[pl.ds(row, 16)][0]
        eps = tp_ref[pl.ds(B, 16)][0]
        v0 = sv_ref[pl.ds(0, 16)][0]

        def ebody(t, z):
            sv = sv_ref[pl.ds(t * 16, 16)]
            r = lanes + t * 16
            e = jnp.where(r < k_row, jnp.exp(sv - v0), jnp.float32(0.0))
            eb_ref[pl.ds(t * 16, 16)] = e
            return z + jnp.sum(e)

        Z = lax.fori_loop(0, NSORT // 16, ebody, jnp.float32(0.0))

        def cbody(t, carry):
            cc, m = carry
            e = eb_ref[pl.ds(t * 16, 16)]
            probs = e / Z
            pb_ref[pl.ds(t * 16, 16)] = probs
            cum = plsc.cumsum(probs) + cc
            r = lanes + t * 16
            keep = jnp.logical_and(r < k_row, (cum - probs) <= p_row)
            m = m + jnp.sum(keep.astype(jnp.int32))
            return jnp.max(cum), m

        _, m = lax.fori_loop(0, NSORT // 16, cbody,
                             (jnp.float32(0.0), jnp.int32(0)))

        qh.wait()

        def rbody(t, best):
            r = lanes + t * 16
            probs = pb_ref[pl.ds(t * 16, 16)]
            qv = plsc.load_gather(row_ref, [si_ref[pl.ds(t * 16, 16)]])
            ratio = jnp.where(r < m, probs / (qv + eps), jnp.float32(0.0))
            rb_ref[pl.ds(t * 16, 16)] = ratio
            return jnp.maximum(best, jnp.max(ratio))

        best = lax.fori_loop(0, NSORT // 16, rbody, jnp.float32(0.0))

        def tbody(t, tok):
            r = lanes + t * 16
            ratio = rb_ref[pl.ds(t * 16, 16)]
            iv = si_ref[pl.ds(t * 16, 16)]
            hit = jnp.logical_and(ratio == best, r < m)
            return jnp.minimum(tok, jnp.min(jnp.where(hit, iv, V)))

        tok = lax.fori_loop(0, NSORT // 16, tbody, jnp.int32(V))
        plsc.store_scatter(tokbuf_ref, [jnp.full((16,), 0, jnp.int32) + j],
                           jnp.zeros((16,), jnp.int32) + tok,
                           mask=lanes == 0)

        @plsc.parallel_loop(0, NVREG, unroll=10)
        def _(i):
            row_ref[pl.ds(i * 16, 16)] = negv

        def wbody(t, _):
            r = lanes + t * 16
            plsc.store_scatter(row_ref, [si_ref[pl.ds(t * 16, 16)]],
                               sv_ref[pl.ds(t * 16, 16)], mask=r < m)
            return 0

        lax.fori_loop(0, NSORT // 16, wbody, 0)
        pltpu.sync_copy(row_ref, out_hbm.at[row])
        return 0

    lax.fori_loop(0, ROWS_PER_W, do_row, 0)
    pltpu.sync_copy(tokbuf_ref, tok_hbm.at[wid])


@jax.jit
def kernel(logits, top_k, top_p, q, eps):
    tpeps = jnp.concatenate(
        [top_p.astype(jnp.float32),
         jnp.full((32,), eps, jnp.float32)])
    topk_pad = jnp.concatenate(
        [top_k.astype(jnp.int32), jnp.zeros((32,), jnp.int32)])
    mesh = plsc.VectorSubcoreMesh(core_axis_name="c", subcore_axis_name="s",
                                  num_cores=NC, num_subcores=NS)
    f = pl.kernel(
        _body,
        out_type=[
            jax.ShapeDtypeStruct((NW, 16), jnp.int32),
            jax.ShapeDtypeStruct((B, V), jnp.float32),
        ],
        mesh=mesh,
        compiler_params=pltpu.CompilerParams(needs_layout_passes=False),
        scratch_types=[
            pltpu.VMEM((V,), jnp.float32),
            pltpu.VMEM((2048,), jnp.int32),
            pltpu.VMEM((CAP + 16,), jnp.int32),
            pltpu.VMEM((CAP + 16,), jnp.int32),
            pltpu.VMEM((NSORT + 16,), jnp.float32),
            pltpu.VMEM((NSORT + 16,), jnp.int32),
            pltpu.VMEM((NSORT,), jnp.float32),
            pltpu.VMEM((NSORT,), jnp.float32),
            pltpu.VMEM((NSORT,), jnp.float32),
            pltpu.VMEM((B + 32,), jnp.int32),
            pltpu.VMEM((B + 32,), jnp.float32),
            pltpu.VMEM((16,), jnp.int32),
            pltpu.VMEM((NBLK + 16,), jnp.int32),
            pltpu.VMEM((144,), jnp.int32),
            pltpu.SemaphoreType.DMA,
        ],
    )
    tok_pad, masked = f(logits, q, topk_pad, tpeps)
    tokens = tok_pad[:, :ROWS_PER_W].reshape(B)
    return tokens, masked


# speedup vs baseline: 1.1502x; 1.0022x over previous
import numpy as np

import jax
import jax.numpy as jnp
from jax import lax
from jax.experimental import pallas as pl
from jax.experimental.pallas import tpu as pltpu
from jax.experimental.pallas import tpu_sc as plsc

B = 128
V = 100000
NVREG = V // 16
NC, NS, L = 2, 16, 16
NW = NC * NS
ROWS_PER_W = B // NW
NEED = 99
CAP = 512
NSORT = 112
BLK = 25
NBLK = NVREG // BLK
NEG = float(jnp.finfo(jnp.float32).min)
HIBIT = np.uint32(0x80000000)


def _key_of(v):
    u = lax.bitcast_convert_type(v, jnp.uint32)
    sa = lax.shift_right_arithmetic(lax.bitcast_convert_type(v, jnp.int32), 31)
    return u ^ (lax.bitcast_convert_type(sa, jnp.uint32) | HIBIT)


def _scan_hist(hist_ref, nbuckets, cg):
    nv = nbuckets // 16
    lanes = lax.iota(jnp.int32, 16)

    def body(i, carry):
        acc, chosen, cnt, found = carry
        t = nv - 1 - i
        h = hist_ref[pl.ds(t * 16, 16)]
        tot = jnp.sum(h)
        crossing = jnp.logical_and(jnp.logical_not(found),
                                   cg + acc + tot >= NEED)
        cum = plsc.cumsum(h)
        suff = tot - cum + h
        cond = (cg + acc + suff) >= NEED
        lane = jnp.max(jnp.where(cond, lanes, -1))
        lane_c = jnp.maximum(lane, 0)
        onlane = lanes == lane_c
        h_at = jnp.max(jnp.where(onlane, h, 0))
        cum_at = jnp.max(jnp.where(onlane, cum, 0))
        acc_new = jnp.where(found, acc,
                            jnp.where(crossing, acc + (tot - cum_at),
                                      acc + tot))
        chosen = jnp.where(crossing, t * 16 + lane_c, chosen)
        cnt = jnp.where(crossing, h_at, cnt)
        found = jnp.logical_or(found, crossing)
        return acc_new, chosen, cnt, found

    acc, chosen, cnt, _ = lax.fori_loop(
        0, nv, body, (jnp.int32(0), jnp.int32(0), jnp.int32(0),
                      jnp.bool_(False)))
    cg_new = cg + acc
    return chosen, cg_new, cg_new + cnt


def _scan_hist_h(hist_ref, tot_ref, nbuckets, cg):
    nv = nbuckets // 16
    lanes = lax.iota(jnp.int32, 16)

    @plsc.parallel_loop(0, nv, unroll=8)
    def _(i):
        h = hist_ref[pl.ds(i * 16, 16)]
        plsc.store_scatter(tot_ref, [jnp.full((16,), 0, jnp.int32) + i],
                           jnp.zeros((16,), jnp.int32) + jnp.sum(h),
                           mask=lanes == 0)

    tv, cgv, _ = _scan_hist(tot_ref, nv, cg)
    h = hist_ref[pl.ds(tv * 16, 16)]
    tot = jnp.sum(h)
    cum = plsc.cumsum(h)
    suff = tot - cum + h
    cond = (cgv + suff) >= NEED
    lane = jnp.maximum(jnp.max(jnp.where(cond, lanes, -1)), 0)
    onlane = lanes == lane
    h_at = jnp.max(jnp.where(onlane, h, 0))
    cum_at = jnp.max(jnp.where(onlane, cum, 0))
    cg_new = cgv + (tot - cum_at)
    return tv * 16 + lane, cg_new, cg_new + h_at


def _zero_hist(hist_ref):
    zeros = jnp.zeros((16,), jnp.int32)

    @plsc.parallel_loop(0, 2048 // 16, unroll=8)
    def _(i):
        hist_ref[pl.ds(i * 16, 16)] = zeros


def _body(logits_hbm, q_hbm, topk_hbm, tpeps_hbm,
          tok_hbm, out_hbm,
          row_ref, hist_ref, ck_ref, ci_ref,
          sv_ref, si_ref, eb_ref, pb_ref, rb_ref,
          tk_ref, tp_ref, tokbuf_ref, bm_ref, tot_ref, sem):
    wid = lax.axis_index("s") * NC + lax.axis_index("c")
    lanes = lax.iota(jnp.int32, 16)
    ones = jnp.ones((16,), jnp.int32)

    pltpu.sync_copy(topk_hbm, tk_ref)
    pltpu.sync_copy(tpeps_hbm, tp_ref)
    tokbuf_ref[pl.ds(0, 16)] = jnp.zeros((16,), jnp.int32)

    negv = jnp.full((16,), NEG, jnp.float32)

    def do_row(j, _):
        row = wid * ROWS_PER_W + j
        pltpu.sync_copy(logits_hbm.at[row], row_ref)
        _zero_hist(hist_ref)

        def p1b(bi, _):
            base = bi * BLK

            @plsc.parallel_loop(0, BLK, unroll=25,
                                carry=jnp.zeros((16,), jnp.uint32))
            def mx(i, bm):
                v = row_ref[pl.ds((base + i) * 16, 16)]
                k = _key_of(v)
                row_ref[pl.ds((base + i) * 16, 16)] = (
                    lax.bitcast_convert_type(k, jnp.float32))
                b = lax.bitcast_convert_type(k >> 21, jnp.int32)
                plsc.addupdate_scatter(hist_ref, [b], ones)
                return jnp.maximum(bm, k)

            bms = jnp.max(lax.bitcast_convert_type(mx ^ HIBIT, jnp.int32))
            plsc.store_scatter(bm_ref,
                               [jnp.full((16,), 0, jnp.int32) + bi],
                               jnp.zeros((16,), jnp.int32) + bms,
                               mask=lanes == 0)
            return 0

        lax.fori_loop(0, NBLK, p1b, 0)
        b0, cg, count_ge = _scan_hist_h(hist_ref, tot_ref, 2048,
                                        jnp.int32(0))
        T = lax.convert_element_type(b0, jnp.uint32) << 21

        def _kld(i):
            return lax.bitcast_convert_type(row_ref[pl.ds(i * 16, 16)],
                                            jnp.uint32)

        def refine(level_shift, level_mask, prev_shift, T, cg):
            _zero_hist(hist_ref)
            Tp = T >> prev_shift
            pmin_s = lax.bitcast_convert_type((Tp << prev_shift) ^ HIBIT,
                                              jnp.int32)

            def rblk(bi, _):
                bms = bm_ref[pl.ds(bi, 16)][0]
                base = bi * BLK

                def go(z):
                    @plsc.parallel_loop(0, BLK, unroll=25)
                    def _(i):
                        k = _kld(base + i)
                        part = (k >> prev_shift) == Tp
                        b = lax.bitcast_convert_type(
                            (k >> level_shift) & jnp.uint32(level_mask),
                            jnp.int32)
                        plsc.addupdate_scatter(hist_ref, [b], ones,
                                               mask=part)
                    return z

                return lax.cond(bms >= pmin_s, go, lambda z: z, 0)

            lax.fori_loop(0, NBLK, rblk, 0)
            bb, cg2, cge = _scan_hist(hist_ref, level_mask + 1, cg)
            T2 = T | (lax.convert_element_type(bb, jnp.uint32) << level_shift)
            return T2, cg2, cge

        def lvl1(args):
            T, cg, _ = args
            return refine(10, 0x7FF, 21, T, cg)

        T, cg, count_ge = lax.cond(count_ge <= CAP,
                                   lambda a: a, lvl1, (T, cg, count_ge))

        def lvl2(args):
            T, cg, _ = args
            return refine(0, 0x3FF, 10, T, cg)

        T, cg, count_ge = lax.cond(count_ge <= CAP,
                                   lambda a: a, lvl2, (T, cg, count_ge))

        Ts = lax.bitcast_convert_type(T ^ HIBIT, jnp.int32)

        def lvlI(args):
            T, cg, _ = args
            _zero_hist(hist_ref)

            def iblk(bi, _):
                bms = bm_ref[pl.ds(bi, 16)][0]
                base = bi * BLK

                def go(z):
                    @plsc.parallel_loop(0, BLK, unroll=25)
                    def _(i):
                        k = _kld(base + i)
                        part = k == T
                        idx = lanes + (base + i) * 16
                        b = lax.shift_right_logical(idx, 8)
                        plsc.addupdate_scatter(hist_ref, [b], ones,
                                               mask=part)
                    return z

                return lax.cond(bms >= Ts, go, lambda z: z, 0)

            lax.fori_loop(0, NBLK, iblk, 0)
            bb, _, _ = _scan_hist(hist_ref, 512, cg)
            return lax.shift_left(bb, 8)

        I = lax.cond(count_ge <= CAP, lambda a: jnp.int32(0), lvlI,
                     (T, cg, count_ge))

        def cblk_fast(bi, ptr):
            bms = bm_ref[pl.ds(bi, 16)][0]
            base = bi * BLK

            def go(p):
                @plsc.parallel_loop(0, BLK, unroll=25, carry=p)
                def cc(i, q):
                    k = _kld(base + i)
                    mk = k >= T
                    plsc.store_compressed(
                        ck_ref.at[pl.ds(q, 16)],
                        lax.bitcast_convert_type(k, jnp.int32), mask=mk)
                    plsc.store_compressed(ci_ref.at[pl.ds(q, 16)],
                                          lanes + (base + i) * 16, mask=mk)
                    return q + plsc.all_reduce_population_count(mk)[0]

                return cc

            return lax.cond(bms >= Ts, go, lambda p: p, ptr)

        def cblk_tie(bi, ptr):
            bms = bm_ref[pl.ds(bi, 16)][0]
            base = bi * BLK

            def go(p):
                @plsc.parallel_loop(0, BLK, unroll=25, carry=p)
                def cc(i, q):
                    k = _kld(base + i)
                    idx = lanes + (base + i) * 16
                    mk = jnp.logical_or(k > T,
                                        jnp.logical_and(k == T, idx >= I))
                    plsc.store_compressed(
                        ck_ref.at[pl.ds(q, 16)],
                        lax.bitcast_convert_type(k, jnp.int32), mask=mk)
                    plsc.store_compressed(ci_ref.at[pl.ds(q, 16)], idx,
                                          mask=mk)
                    return q + plsc.all_reduce_population_count(mk)[0]

                return cc

            return lax.cond(bms >= Ts, go, lambda p: p, ptr)

        n = lax.cond(
            I == 0,
            lambda z: lax.fori_loop(0, NBLK, cblk_fast, z),
            lambda z: lax.fori_loop(0, NBLK, cblk_tie, z),
            jnp.int32(0))

        qh = pltpu.async_copy(q_hbm.at[row], row_ref, sem)

        def sinit(t, _):
            sv_ref[pl.ds(t * 16, 16)] = jnp.full((16,), NEG, jnp.float32)
            si_ref[pl.ds(t * 16, 16)] = jnp.zeros((16,), jnp.int32)
            return 0

        lax.fori_loop(0, NSORT // 16, sinit, 0)

        nv_c = lax.shift_right_logical(n + 15, 4)

        def rank_one(i, _):
            ki = lax.convert_element_type(ck_ref[pl.ds(i, 16)][0], jnp.uint32)
            ii = ci_ref[pl.ds(i, 16)][0]

            def rin(jv, r):
                kv = lax.convert_element_type(ck_ref[pl.ds(jv * 16, 16)],
                                              jnp.uint32)
                iv = ci_ref[pl.ds(jv * 16, 16)]
                ok = (jv * 16 + lanes) < n
                gt = jnp.logical_or(kv > ki,
                                    jnp.logical_and(kv == ki, iv > ii))
                return r + plsc.all_reduce_population_count(
                    jnp.logical_and(gt, ok))

            rank_v = lax.fori_loop(0, nv_c, rin, jnp.zeros((16,), jnp.int32))
            rank = rank_v[0]
            neg = (ki >> 31) == 0
            bits = jnp.where(neg, ~ki, ki ^ HIBIT)
            val = lax.bitcast_convert_type(bits, jnp.float32)
            lane0 = jnp.logical_and(lanes == 0, rank < NEED)
            slot_v = jnp.full((16,), 0, jnp.int32) + rank
            plsc.store_scatter(sv_ref, [slot_v],
                               jnp.zeros((16,), jnp.float32) + val,
                               mask=lane0)
            plsc.store_scatter(si_ref, [slot_v],
                               jnp.zeros((16,), jnp.int32) + ii,
                               mask=lane0)
            return 0

        lax.fori_loop(0, n, rank_one, 0)

        k_row = tk_ref[pl.ds(row, 16)][0]
        p_row = tp_ref[pl.ds(row, 16)][0]
        eps = tp_ref[pl.ds(B, 16)][0]
        v0 = sv_ref[pl.ds(0, 16)][0]

        def ebody(t, z):
            sv = sv_ref[pl.ds(t * 16, 16)]
            r = lanes + t * 16
            e = jnp.where(r < k_row, jnp.exp(sv - v0), jnp.float32(0.0))
            eb_ref[pl.ds(t * 16, 16)] = e
            return z + jnp.sum(e)

        Z = lax.fori_loop(0, NSORT // 16, ebody, jnp.float32(0.0))

        def cbody(t, carry):
            cc, m = carry
            e = eb_ref[pl.ds(t * 16, 16)]
            probs = e / Z
            pb_ref[pl.ds(t * 16, 16)] = probs
            cum = plsc.cumsum(probs) + cc
            r = lanes + t * 16
            keep = jnp.logical_and(r < k_row, (cum - probs) <= p_row)
            m = m + jnp.sum(keep.astype(jnp.int32))
            return jnp.max(cum), m

        _, m = lax.fori_loop(0, NSORT // 16, cbody,
                             (jnp.float32(0.0), jnp.int32(0)))

        qh.wait()

        def rbody(t, best):
            r = lanes + t * 16
            probs = pb_ref[pl.ds(t * 16, 16)]
            qv = plsc.load_gather(row_ref, [si_ref[pl.ds(t * 16, 16)]])
            ratio = jnp.where(r < m, probs / (qv + eps), jnp.float32(0.0))
            rb_ref[pl.ds(t * 16, 16)] = ratio
            return jnp.maximum(best, jnp.max(ratio))

        best = lax.fori_loop(0, NSORT // 16, rbody, jnp.float32(0.0))

        def tbody(t, tok):
            r = lanes + t * 16
            ratio = rb_ref[pl.ds(t * 16, 16)]
            iv = si_ref[pl.ds(t * 16, 16)]
            hit = jnp.logical_and(ratio == best, r < m)
            return jnp.minimum(tok, jnp.min(jnp.where(hit, iv, V)))

        tok = lax.fori_loop(0, NSORT // 16, tbody, jnp.int32(V))
        plsc.store_scatter(tokbuf_ref, [jnp.full((16,), 0, jnp.int32) + j],
                           jnp.zeros((16,), jnp.int32) + tok,
                           mask=lanes == 0)

        @plsc.parallel_loop(0, NVREG, unroll=10)
        def _(i):
            row_ref[pl.ds(i * 16, 16)] = negv

        def wbody(t, _):
            r = lanes + t * 16
            plsc.store_scatter(row_ref, [si_ref[pl.ds(t * 16, 16)]],
                               sv_ref[pl.ds(t * 16, 16)], mask=r < m)
            return 0

        lax.fori_loop(0, NSORT // 16, wbody, 0)
        pltpu.sync_copy(row_ref, out_hbm.at[row])
        return 0

    lax.fori_loop(0, ROWS_PER_W, do_row, 0)
    pltpu.sync_copy(tokbuf_ref, tok_hbm.at[wid])


@jax.jit
def kernel(logits, top_k, top_p, q, eps):
    tpeps = jnp.concatenate(
        [top_p.astype(jnp.float32),
         jnp.full((32,), eps, jnp.float32)])
    topk_pad = jnp.concatenate(
        [top_k.astype(jnp.int32), jnp.zeros((32,), jnp.int32)])
    mesh = plsc.VectorSubcoreMesh(core_axis_name="c", subcore_axis_name="s",
                                  num_cores=NC, num_subcores=NS)
    f = pl.kernel(
        _body,
        out_type=[
            jax.ShapeDtypeStruct((NW, 16), jnp.int32),
            jax.ShapeDtypeStruct((B, V), jnp.float32),
        ],
        mesh=mesh,
        compiler_params=pltpu.CompilerParams(needs_layout_passes=False),
        scratch_types=[
            pltpu.VMEM((V,), jnp.float32),
            pltpu.VMEM((2048,), jnp.int32),
            pltpu.VMEM((CAP + 16,), jnp.int32),
            pltpu.VMEM((CAP + 16,), jnp.int32),
            pltpu.VMEM((NSORT + 16,), jnp.float32),
            pltpu.VMEM((NSORT + 16,), jnp.int32),
            pltpu.VMEM((NSORT,), jnp.float32),
            pltpu.VMEM((NSORT,), jnp.float32),
            pltpu.VMEM((NSORT,), jnp.float32),
            pltpu.VMEM((B + 32,), jnp.int32),
            pltpu.VMEM((B + 32,), jnp.float32),
            pltpu.VMEM((16,), jnp.int32),
            pltpu.VMEM((NBLK + 16,), jnp.int32),
            pltpu.VMEM((144,), jnp.int32),
            pltpu.SemaphoreType.DMA,
        ],
    )
    tok_pad, masked = f(logits, q, topk_pad, tpeps)
    tokens = tok_pad[:, :ROWS_PER_W].reshape(B)
    return tokens, masked
